# baseline (speedup 1.0000x reference)
import functools

import jax
import jax.numpy as jnp
from jax import lax
from jax.experimental import pallas as pl
from jax.experimental.pallas import tpu as pltpu
from jax.experimental.pallas import tpu_sc as plsc

B, S, D = 4, 4096, 2048
N = 128
TOTAL = B * N
L = 16
NC, NS = 2, 16
NW = NC * NS
BPW = TOTAL // NW
HB = BPW // 2
SLOT = 16
CHUNKS = D // L
UNROLL = 8

_mesh = plsc.VectorSubcoreMesh(core_axis_name="c", subcore_axis_name="s")


@functools.partial(
    pl.kernel,
    mesh=_mesh,
    out_type=jax.ShapeDtypeStruct((TOTAL, D), jnp.float32),
    scratch_types=(
        [pltpu.VMEM((BPW,), jnp.int32),
         pltpu.VMEM((BPW,), jnp.int32),
         pltpu.VMEM((BPW * SLOT,), jnp.int32)]
        + [pltpu.VMEM((1, D), jnp.float32) for _ in range(BPW)]
        + [pltpu.SemaphoreType.DMA] * 5
    ),
)
def _gather_pool(wv_hbm, ids_hbm, mask_hbm, out_hbm, *scratch):
    idx_v, mask_v, slot_v = scratch[0], scratch[1], scratch[2]
    bufs = scratch[3:3 + BPW]
    semi, semm, semA, semB, semw = scratch[3 + BPW:]

    wid = lax.axis_index("s") * NC + lax.axis_index("c")
    base = wid * BPW
    ci = pltpu.async_copy(ids_hbm.at[pl.ds(base, BPW)], idx_v, semi)
    cm = pltpu.async_copy(mask_hbm.at[pl.ds(base, BPW)], mask_v, semm)
    ci.wait()
    cm.wait()
    boff = (base // N) * S
    idx_reg = idx_v[...] + boff
    for r in range(BPW):
        slot_v[pl.ds(r * SLOT, L)] = idx_reg.at[
            jnp.full((L,), r, jnp.int32)].get(mode="promise_in_bounds")

    mask_reg = mask_v[...]
    zero = jnp.zeros((L,), jnp.float32)
    halves = [(semA, 0), (semB, HB)]

    for sem, r0 in halves:
        for r in range(HB):
            g = r0 + r

            @pl.when(mask_reg[g] != 0)
            def _fetch(sem=sem, g=g):
                pltpu.async_copy(wv_hbm.at[slot_v.at[pl.ds(g * SLOT, 1)]],
                                 bufs[g], sem)

            @pl.when(mask_reg[g] == 0)
            def _zero(g=g):
                def col_body(j, _):
                    for u in range(UNROLL):
                        bufs[g][0, pl.ds((j * UNROLL + u) * L, L)] = zero
                    return 0

                lax.fori_loop(0, CHUNKS // UNROLL, col_body, 0)

    writes = []
    for sem, r0 in halves:
        for r in range(HB):
            g = r0 + r

            @pl.when(mask_reg[g] != 0)
            def _drain(sem=sem, g=g):
                pltpu.make_async_copy(wv_hbm.at[pl.ds(0, 1)],
                                      bufs[g], sem).wait()

        for r in range(HB):
            g = r0 + r
            writes.append(
                pltpu.async_copy(bufs[g],
                                 out_hbm.at[pl.ds(base + g, 1)], semw))
    for w in writes:
        w.wait()


def kernel(word_vectors, sent_rep_token_ids, sent_rep_mask):
    wv2d = word_vectors.reshape(B * S, D)
    ids = sent_rep_token_ids.reshape(TOTAL)
    msk = sent_rep_mask.reshape(TOTAL)
    out = _gather_pool(wv2d, ids, msk)
    return out.reshape(B, N, D), sent_rep_mask
